# Initial kernel scaffold; baseline (speedup 1.0000x reference)
import jax
import jax.numpy as jnp
from jax.experimental import pallas as pl


def kernel(scores):
    raise NotImplementedError("write your pallas kernel here")


# speedup vs baseline: 2.3566x; 2.3566x over previous
import jax
import jax.numpy as jnp
from jax.experimental import pallas as pl

_NMS_RADIUS = 3
_ITERATIONS = 2
_NEG_INF = float("-inf")


def _shift(x, d, axis):
    n = x.shape[axis]
    if d == 0:
        return x
    if axis == 0:
        pad = jnp.full((abs(d), x.shape[1]), _NEG_INF, x.dtype)
        if d > 0:
            return jnp.concatenate([pad, x[: n - d, :]], axis=0)
        return jnp.concatenate([x[-d:, :], pad], axis=0)
    pad = jnp.full((x.shape[0], abs(d)), _NEG_INF, x.dtype)
    if d > 0:
        return jnp.concatenate([pad, x[:, : n - d]], axis=1)
    return jnp.concatenate([x[:, -d:], pad], axis=1)


def _maxpool1d(x, axis):
    t = x
    for d in (1, 2, 3, -1, -2, -3):
        t = jnp.maximum(t, _shift(x, d, axis))
    return t


def _maxpool(x):
    return _maxpool1d(_maxpool1d(x, 1), 0)


def _nms_kernel(s_ref, o_ref):
    x = s_ref[0, 0]
    max_mask = x == _maxpool(x)
    for _ in range(_ITERATIONS):
        supp_mask = _maxpool(max_mask.astype(jnp.float32)) > 0
        supp_scores = jnp.where(supp_mask, 0.0, x)
        new_max = (supp_scores == _maxpool(supp_scores)) & (~supp_mask)
        max_mask = max_mask | new_max
    o_ref[0, 0] = jnp.where(max_mask, x, 0.0)


def kernel(scores):
    b, c, h, w = scores.shape
    return pl.pallas_call(
        _nms_kernel,
        grid=(b * c,),
        in_specs=[pl.BlockSpec((1, 1, h, w), lambda i: (i, 0, 0, 0))],
        out_specs=pl.BlockSpec((1, 1, h, w), lambda i: (i, 0, 0, 0)),
        out_shape=jax.ShapeDtypeStruct(scores.shape, scores.dtype),
    )(scores)


# speedup vs baseline: 3.6540x; 1.5506x over previous
import jax
import jax.numpy as jnp
from jax.experimental import pallas as pl

_NMS_RADIUS = 3
_ITERATIONS = 2
_NEG_INF = float("-inf")


def _shift(x, d, axis):
    n = x.shape[axis]
    if d == 0:
        return x
    if axis == 0:
        pad = jnp.full((abs(d), x.shape[1]), _NEG_INF, x.dtype)
        if d > 0:
            return jnp.concatenate([pad, x[: n - d, :]], axis=0)
        return jnp.concatenate([x[-d:, :], pad], axis=0)
    pad = jnp.full((x.shape[0], abs(d)), _NEG_INF, x.dtype)
    if d > 0:
        return jnp.concatenate([pad, x[:, : n - d]], axis=1)
    return jnp.concatenate([x[:, -d:], pad], axis=1)


def _maxpool1d(x, axis):
    s = jnp.maximum(x, _shift(x, 1, axis))
    s = jnp.maximum(s, _shift(s, 2, axis))
    t = jnp.maximum(x, _shift(x, -1, axis))
    t = jnp.maximum(t, _shift(t, -2, axis))
    return jnp.maximum(s, t)


def _maxpool(x):
    return _maxpool1d(_maxpool1d(x, 1), 0)


def _nms_kernel(s_ref, o_ref):
    x = s_ref[0, 0]
    max_mask = x == _maxpool(x)
    for _ in range(_ITERATIONS):
        supp_mask = _maxpool(max_mask.astype(jnp.float32)) > 0
        supp_scores = jnp.where(supp_mask, 0.0, x)
        new_max = (supp_scores == _maxpool(supp_scores)) & (~supp_mask)
        max_mask = max_mask | new_max
    o_ref[0, 0] = jnp.where(max_mask, x, 0.0)


def kernel(scores):
    b, c, h, w = scores.shape
    return pl.pallas_call(
        _nms_kernel,
        grid=(b * c,),
        in_specs=[pl.BlockSpec((1, 1, h, w), lambda i: (i, 0, 0, 0))],
        out_specs=pl.BlockSpec((1, 1, h, w), lambda i: (i, 0, 0, 0)),
        out_shape=jax.ShapeDtypeStruct(scores.shape, scores.dtype),
    )(scores)


# speedup vs baseline: 3.8566x; 1.0554x over previous
import jax
import jax.numpy as jnp
from jax.experimental import pallas as pl

_NMS_RADIUS = 3
_ITERATIONS = 2
_NEG_INF = float("-inf")


def _shift(x, d, axis):
    n = x.shape[axis]
    if d == 0:
        return x
    if axis == 0:
        pad = jnp.full((abs(d), x.shape[1]), _NEG_INF, x.dtype)
        if d > 0:
            return jnp.concatenate([pad, x[: n - d, :]], axis=0)
        return jnp.concatenate([x[-d:, :], pad], axis=0)
    pad = jnp.full((x.shape[0], abs(d)), _NEG_INF, x.dtype)
    if d > 0:
        return jnp.concatenate([pad, x[:, : n - d]], axis=1)
    return jnp.concatenate([x[:, -d:], pad], axis=1)


def _maxpool1d(x, axis):
    s = jnp.maximum(x, _shift(x, 1, axis))
    s = jnp.maximum(s, _shift(s, 2, axis))
    t = jnp.maximum(x, _shift(x, -1, axis))
    t = jnp.maximum(t, _shift(t, -2, axis))
    return jnp.maximum(s, t)


def _maxpool(x):
    return _maxpool1d(_maxpool1d(x, 1), 0)


def _nms_kernel(s_ref, o_ref):
    x = s_ref[0, 0]
    max_mask = x == _maxpool(x)
    for _ in range(_ITERATIONS):
        supp_mask = _maxpool(max_mask.astype(jnp.bfloat16)) > 0
        supp_scores = jnp.where(supp_mask, 0.0, x)
        new_max = (supp_scores == _maxpool(supp_scores)) & (~supp_mask)
        max_mask = max_mask | new_max
    o_ref[0, 0] = jnp.where(max_mask, x, 0.0)


def kernel(scores):
    b, c, h, w = scores.shape
    return pl.pallas_call(
        _nms_kernel,
        grid=(b * c,),
        in_specs=[pl.BlockSpec((1, 1, h, w), lambda i: (i, 0, 0, 0))],
        out_specs=pl.BlockSpec((1, 1, h, w), lambda i: (i, 0, 0, 0)),
        out_shape=jax.ShapeDtypeStruct(scores.shape, scores.dtype),
    )(scores)
